# baseline (speedup 1.0000x reference)
import functools

import jax
import jax.numpy as jnp
from jax import lax
from jax.experimental import pallas as pl
from jax.experimental.pallas import tpu as pltpu
from jax.experimental.pallas import tpu_sc as plsc

B = 1024
CTRL = 1024
D = 128
M = 100000
K = 8
MBLK = 1024
NB = (M + MBLK - 1) // MBLK
NEG = -3.0e38


def _qproj_body(x_ref, wq_ref, bq_ref, o_ref):
    q = jnp.dot(x_ref[...], wq_ref[...], preferred_element_type=jnp.float32)
    q = q + bq_ref[...]
    n = jnp.sqrt(jnp.sum(q * q, axis=1, keepdims=True))
    o_ref[...] = q / jnp.maximum(n, 1e-12)


def _qproj(query_input, Wq, bq2):
    return pl.pallas_call(
        _qproj_body,
        grid=(4,),
        in_specs=[
            pl.BlockSpec((B // 4, CTRL), lambda i: (i, 0)),
            pl.BlockSpec((CTRL, D), lambda i: (0, 0)),
            pl.BlockSpec((1, D), lambda i: (0, 0)),
        ],
        out_specs=pl.BlockSpec((B // 4, D), lambda i: (i, 0)),
        out_shape=jax.ShapeDtypeStruct((B, D), jnp.float32),
    )(query_input, Wq, bq2)


BIGI = 2**30


def _extract_topk(vals, idxs, nrounds):
    tv, ti = [], []
    cur = vals
    for _ in range(nrounds):
        m = jnp.max(cur, axis=1, keepdims=True)
        hit = cur >= m
        ii = jnp.min(jnp.where(hit, idxs, BIGI), axis=1, keepdims=True)
        cur = jnp.where(hit & (idxs == ii), NEG, cur)
        tv.append(m)
        ti.append(ii)
    return jnp.concatenate(tv, axis=1), jnp.concatenate(ti, axis=1)


NG = 8
GW = MBLK // NG
NR = 8


def _round(s_ref, mg_ref, cv_ref, ci_ref, cont_ref, accv_ref, r, j):
    s = s_ref[...]
    mgold = mg_ref[...]
    slnews, iis, mgnews = [], [], []
    for g in range(NG):
        sl = s[:, g * GW:(g + 1) * GW]
        gcol_g = (j * MBLK + g * GW
                  + lax.broadcasted_iota(jnp.int32, (B, GW), 1))
        mg_g = mgold[:, g:g + 1]
        ii_g = jnp.min(jnp.where(sl >= mg_g, gcol_g, BIGI),
                       axis=1, keepdims=True)
        sl = jnp.where(gcol_g == ii_g, NEG, sl)
        slnews.append(sl)
        iis.append(ii_g)
        mgnews.append(jnp.max(sl, axis=1, keepdims=True))
    s_ref[...] = jnp.concatenate(slnews, axis=1)
    mgnew = jnp.concatenate(mgnews, axis=1)
    mg_ref[...] = mgnew
    cv_ref[:, r * NG:(r + 1) * NG] = mgold
    ci_ref[:, r * NG:(r + 1) * NG] = jnp.concatenate(iis, axis=1)
    gm = jnp.max(mgnew, axis=1, keepdims=True)
    cont_ref[0] = jnp.any(gm >= accv_ref[:, 7:8]).astype(jnp.int32)


def _simstopk_body(qn_ref, k_ref, iw_ref, idx_ref,
                   accv_ref, acci_ref, s_ref, mg_ref, cv_ref, ci_ref,
                   cont_ref):
    j = pl.program_id(0)
    nb = pl.num_programs(0)

    @pl.when(j == 0)
    def _():
        accv_ref[...] = jnp.full((B, K), NEG, jnp.float32)
        acci_ref[...] = jnp.full((B, K), -1, jnp.int32)

    kblk = k_ref[...]
    ss = jnp.sum(kblk * kblk, axis=1, keepdims=True)
    kn = kblk / jnp.maximum(jnp.sqrt(ss), 1e-12)
    s = lax.dot_general(qn_ref[...], kn, (((1,), (1,)), ((), ())),
                        preferred_element_type=jnp.float32)
    s = s * iw_ref[0]
    gcol = j * MBLK + lax.broadcasted_iota(jnp.int32, (B, MBLK), 1)
    s = jnp.where(gcol < M, s, NEG)
    s_ref[...] = s
    mg_ref[...] = jnp.concatenate(
        [jnp.max(s[:, g * GW:(g + 1) * GW], axis=1, keepdims=True)
         for g in range(NG)], axis=1)
    cv_ref[...] = jnp.full((B, NR * NG), NEG, jnp.float32)
    ci_ref[...] = jnp.full((B, NR * NG), -1, jnp.int32)

    _round(s_ref, mg_ref, cv_ref, ci_ref, cont_ref, accv_ref, 0, j)
    for r in range(1, NR):
        @pl.when(cont_ref[0] == 1)
        def _(r=r):
            _round(s_ref, mg_ref, cv_ref, ci_ref, cont_ref, accv_ref, r, j)

    cand_v = jnp.concatenate([accv_ref[...], cv_ref[...]], axis=1)
    cand_i = jnp.concatenate([acci_ref[...], ci_ref[...]], axis=1)
    new_v, new_i = _extract_topk(cand_v, cand_i, K)
    accv_ref[...] = new_v
    acci_ref[...] = new_i

    @pl.when(j == nb - 1)
    def _():
        idx_ref[...] = new_i


def _simstopk(qn, memory_keys, iw3):
    return pl.pallas_call(
        _simstopk_body,
        grid=(NB,),
        in_specs=[
            pl.BlockSpec((B, D), lambda j: (0, 0)),
            pl.BlockSpec((MBLK, D), lambda j: (j, 0)),
            pl.BlockSpec((1, 1, MBLK), lambda j: (j, 0, 0)),
        ],
        out_specs=pl.BlockSpec((B, K), lambda j: (0, 0)),
        out_shape=jax.ShapeDtypeStruct((B, K), jnp.int32),
        scratch_shapes=[
            pltpu.VMEM((B, K), jnp.float32),
            pltpu.VMEM((B, K), jnp.int32),
            pltpu.VMEM((B, MBLK), jnp.float32),
            pltpu.VMEM((B, NG), jnp.float32),
            pltpu.VMEM((B, NR * NG), jnp.float32),
            pltpu.VMEM((B, NR * NG), jnp.int32),
            pltpu.SMEM((1,), jnp.int32),
        ],
        compiler_params=pltpu.CompilerParams(
            dimension_semantics=("arbitrary",)),
    )(qn, memory_keys, iw3)


def _sc_gather(memory_values, idx_flat):
    info = plsc.get_sparse_core_info()
    nw = info.num_cores * info.num_subcores
    n_idx = B * K
    b_per_w = n_idx // nw
    ch = min(128, b_per_w)
    nch = b_per_w // ch
    mesh = plsc.VectorSubcoreMesh(core_axis_name="c", subcore_axis_name="s")

    @functools.partial(
        pl.kernel,
        mesh=mesh,
        out_type=jax.ShapeDtypeStruct((n_idx, D), jnp.float32),
        scratch_types=[
            pltpu.VMEM((ch,), jnp.int32),
            pltpu.VMEM((ch, D), jnp.float32),
            pltpu.SemaphoreType.DMA,
        ],
    )
    def _gather(table_hbm, idx_hbm, out_hbm, idx_v, rows_v, sem):
        wid = lax.axis_index("s") * info.num_cores + lax.axis_index("c")
        base = wid * b_per_w
        for c in range(nch):
            off = base + c * ch
            pltpu.sync_copy(idx_hbm.at[pl.ds(off, ch)], idx_v)
            pltpu.async_copy(table_hbm.at[idx_v], rows_v, sem).wait()
            pltpu.sync_copy(rows_v, out_hbm.at[pl.ds(off, ch)])

    return _gather(memory_values, idx_flat)


def _attn_body(r_ref, wa_ref, ba_ref, msk_ref, wc_ref, bc_ref, o_ref):
    r = r_ref[...]
    wa = wa_ref[...]
    logits = jnp.sum(r * wa[None, :, :], axis=2)
    logits = logits + ba_ref[...] + msk_ref[...]
    mx = jnp.max(logits, axis=1, keepdims=True)
    e = jnp.exp(logits - mx)
    attn = e / jnp.sum(e, axis=1, keepdims=True)
    comb = jnp.sum(r * attn[:, :, None], axis=1)
    o_ref[...] = jnp.dot(comb, wc_ref[...],
                         preferred_element_type=jnp.float32) + bc_ref[...]


def _attn(r, wa2, ba2, msk, Wc, bc2):
    bb = B // 4
    return pl.pallas_call(
        _attn_body,
        grid=(4,),
        in_specs=[
            pl.BlockSpec((bb, K, D), lambda i: (i, 0, 0)),
            pl.BlockSpec((1, D), lambda i: (0, 0)),
            pl.BlockSpec((1, 1), lambda i: (0, 0)),
            pl.BlockSpec((1, K), lambda i: (0, 0)),
            pl.BlockSpec((D, D), lambda i: (0, 0)),
            pl.BlockSpec((1, D), lambda i: (0, 0)),
        ],
        out_specs=pl.BlockSpec((bb, D), lambda i: (i, 0)),
        out_shape=jax.ShapeDtypeStruct((B, D), jnp.float32),
    )(r, wa2, ba2, msk, Wc, bc2)


def kernel(query_input, memory_keys, memory_values, importance_weights,
           Wq, bq, Wa, ba, Wc, bc, top_k):
    qn = _qproj(query_input, Wq, bq.reshape(1, D))
    iw3 = jnp.pad(importance_weights, (0, NB * MBLK - M)).reshape(NB, 1, MBLK)
    top_idx = _simstopk(qn, memory_keys, iw3)
    retrieved = _sc_gather(memory_values, top_idx.reshape(-1))
    r = retrieved.reshape(B, K, D)
    msk = jnp.where(jnp.arange(K) < top_k, 0.0, -1e30).astype(
        jnp.float32).reshape(1, K)
    return _attn(r, Wa.reshape(1, D), ba.reshape(1, 1), msk,
                 Wc, bc.reshape(1, D))


# speedup vs baseline: 1.3093x; 1.3093x over previous
import functools

import jax
import jax.numpy as jnp
from jax import lax
from jax.experimental import pallas as pl
from jax.experimental.pallas import tpu as pltpu
from jax.experimental.pallas import tpu_sc as plsc

B = 1024
CTRL = 1024
D = 128
M = 100000
K = 8
MBLK = 1024
NB = (M + MBLK - 1) // MBLK
NEG = -3.0e38


def _qproj_body(x_ref, wq_ref, bq_ref, o_ref):
    q = jnp.dot(x_ref[...], wq_ref[...], preferred_element_type=jnp.float32)
    q = q + bq_ref[...]
    n = jnp.sqrt(jnp.sum(q * q, axis=1, keepdims=True))
    o_ref[...] = q / jnp.maximum(n, 1e-12)


def _qproj(query_input, Wq, bq2):
    return pl.pallas_call(
        _qproj_body,
        grid=(4,),
        in_specs=[
            pl.BlockSpec((B // 4, CTRL), lambda i: (i, 0)),
            pl.BlockSpec((CTRL, D), lambda i: (0, 0)),
            pl.BlockSpec((1, D), lambda i: (0, 0)),
        ],
        out_specs=pl.BlockSpec((B // 4, D), lambda i: (i, 0)),
        out_shape=jax.ShapeDtypeStruct((B, D), jnp.float32),
    )(query_input, Wq, bq2)


BIGI = 2**30


def _extract_topk(vals, idxs, nrounds):
    tv, ti = [], []
    cur = vals
    for _ in range(nrounds):
        m = jnp.max(cur, axis=1, keepdims=True)
        hit = cur >= m
        ii = jnp.min(jnp.where(hit, idxs, BIGI), axis=1, keepdims=True)
        cur = jnp.where(hit & (idxs == ii), NEG, cur)
        tv.append(m)
        ti.append(ii)
    return jnp.concatenate(tv, axis=1), jnp.concatenate(ti, axis=1)


NG = 8
GW = MBLK // NG
NR = 8


def _round(s_ref, mg_ref, cv_ref, ci_ref, cont_ref, accv_ref, r, j):
    s = s_ref[...]
    mgold = mg_ref[...]
    slnews, iis, mgnews = [], [], []
    for g in range(NG):
        sl = s[:, g * GW:(g + 1) * GW]
        gcol_g = (j * MBLK + g * GW
                  + lax.broadcasted_iota(jnp.int32, (B, GW), 1))
        mg_g = mgold[:, g:g + 1]
        ii_g = jnp.min(jnp.where(sl >= mg_g, gcol_g, BIGI),
                       axis=1, keepdims=True)
        sl = jnp.where(gcol_g == ii_g, NEG, sl)
        slnews.append(sl)
        iis.append(ii_g)
        mgnews.append(jnp.max(sl, axis=1, keepdims=True))
    s_ref[...] = jnp.concatenate(slnews, axis=1)
    mgnew = jnp.concatenate(mgnews, axis=1)
    mg_ref[...] = mgnew
    cv_ref[:, r * NG:(r + 1) * NG] = mgold
    ci_ref[:, r * NG:(r + 1) * NG] = jnp.concatenate(iis, axis=1)
    gm = jnp.max(mgnew, axis=1, keepdims=True)
    cont_ref[0] = jnp.int32(0) * jnp.any(gm >= accv_ref[:, 7:8]).astype(jnp.int32)


def _simstopk_body(qn_ref, k_ref, iw_ref, idx_ref,
                   accv_ref, acci_ref, s_ref, mg_ref, cv_ref, ci_ref,
                   cont_ref):
    j = pl.program_id(0)
    nb = pl.num_programs(0)

    @pl.when(j == 0)
    def _():
        accv_ref[...] = jnp.full((B, K), NEG, jnp.float32)
        acci_ref[...] = jnp.full((B, K), -1, jnp.int32)

    kblk = k_ref[...]
    ss = jnp.sum(kblk * kblk, axis=1, keepdims=True)
    kn = kblk / jnp.maximum(jnp.sqrt(ss), 1e-12)
    s = lax.dot_general(qn_ref[...], kn, (((1,), (1,)), ((), ())),
                        preferred_element_type=jnp.float32)
    s = s * iw_ref[0]
    gcol = j * MBLK + lax.broadcasted_iota(jnp.int32, (B, MBLK), 1)
    s = jnp.where(gcol < M, s, NEG)
    s_ref[...] = s
    mg_ref[...] = jnp.concatenate(
        [jnp.max(s[:, g * GW:(g + 1) * GW], axis=1, keepdims=True)
         for g in range(NG)], axis=1)
    cv_ref[...] = jnp.full((B, NR * NG), NEG, jnp.float32)
    ci_ref[...] = jnp.full((B, NR * NG), -1, jnp.int32)

    _round(s_ref, mg_ref, cv_ref, ci_ref, cont_ref, accv_ref, 0, j)
    for r in range(1, NR):
        @pl.when(cont_ref[0] == 1)
        def _(r=r):
            _round(s_ref, mg_ref, cv_ref, ci_ref, cont_ref, accv_ref, r, j)

    cand_v = jnp.concatenate([accv_ref[...], cv_ref[...]], axis=1)
    cand_i = jnp.concatenate([acci_ref[...], ci_ref[...]], axis=1)
    new_v, new_i = _extract_topk(cand_v, cand_i, K)
    accv_ref[...] = new_v
    acci_ref[...] = new_i

    @pl.when(j == nb - 1)
    def _():
        idx_ref[...] = new_i


def _simstopk(qn, memory_keys, iw3):
    return pl.pallas_call(
        _simstopk_body,
        grid=(NB,),
        in_specs=[
            pl.BlockSpec((B, D), lambda j: (0, 0)),
            pl.BlockSpec((MBLK, D), lambda j: (j, 0)),
            pl.BlockSpec((1, 1, MBLK), lambda j: (j, 0, 0)),
        ],
        out_specs=pl.BlockSpec((B, K), lambda j: (0, 0)),
        out_shape=jax.ShapeDtypeStruct((B, K), jnp.int32),
        scratch_shapes=[
            pltpu.VMEM((B, K), jnp.float32),
            pltpu.VMEM((B, K), jnp.int32),
            pltpu.VMEM((B, MBLK), jnp.float32),
            pltpu.VMEM((B, NG), jnp.float32),
            pltpu.VMEM((B, NR * NG), jnp.float32),
            pltpu.VMEM((B, NR * NG), jnp.int32),
            pltpu.SMEM((1,), jnp.int32),
        ],
        compiler_params=pltpu.CompilerParams(
            dimension_semantics=("arbitrary",)),
    )(qn, memory_keys, iw3)


def _sc_gather(memory_values, idx_flat):
    info = plsc.get_sparse_core_info()
    nw = info.num_cores * info.num_subcores
    n_idx = B * K
    b_per_w = n_idx // nw
    ch = min(128, b_per_w)
    nch = b_per_w // ch
    mesh = plsc.VectorSubcoreMesh(core_axis_name="c", subcore_axis_name="s")

    @functools.partial(
        pl.kernel,
        mesh=mesh,
        out_type=jax.ShapeDtypeStruct((n_idx, D), jnp.float32),
        scratch_types=[
            pltpu.VMEM((ch,), jnp.int32),
            pltpu.VMEM((ch, D), jnp.float32),
            pltpu.SemaphoreType.DMA,
        ],
    )
    def _gather(table_hbm, idx_hbm, out_hbm, idx_v, rows_v, sem):
        wid = lax.axis_index("s") * info.num_cores + lax.axis_index("c")
        base = wid * b_per_w
        for c in range(nch):
            off = base + c * ch
            pltpu.sync_copy(idx_hbm.at[pl.ds(off, ch)], idx_v)
            pltpu.async_copy(table_hbm.at[idx_v], rows_v, sem).wait()
            pltpu.sync_copy(rows_v, out_hbm.at[pl.ds(off, ch)])

    return _gather(memory_values, idx_flat)


def _attn_body(r_ref, wa_ref, ba_ref, msk_ref, wc_ref, bc_ref, o_ref):
    r = r_ref[...]
    wa = wa_ref[...]
    logits = jnp.sum(r * wa[None, :, :], axis=2)
    logits = logits + ba_ref[...] + msk_ref[...]
    mx = jnp.max(logits, axis=1, keepdims=True)
    e = jnp.exp(logits - mx)
    attn = e / jnp.sum(e, axis=1, keepdims=True)
    comb = jnp.sum(r * attn[:, :, None], axis=1)
    o_ref[...] = jnp.dot(comb, wc_ref[...],
                         preferred_element_type=jnp.float32) + bc_ref[...]


def _attn(r, wa2, ba2, msk, Wc, bc2):
    bb = B // 4
    return pl.pallas_call(
        _attn_body,
        grid=(4,),
        in_specs=[
            pl.BlockSpec((bb, K, D), lambda i: (i, 0, 0)),
            pl.BlockSpec((1, D), lambda i: (0, 0)),
            pl.BlockSpec((1, 1), lambda i: (0, 0)),
            pl.BlockSpec((1, K), lambda i: (0, 0)),
            pl.BlockSpec((D, D), lambda i: (0, 0)),
            pl.BlockSpec((1, D), lambda i: (0, 0)),
        ],
        out_specs=pl.BlockSpec((bb, D), lambda i: (i, 0)),
        out_shape=jax.ShapeDtypeStruct((B, D), jnp.float32),
    )(r, wa2, ba2, msk, Wc, bc2)


def kernel(query_input, memory_keys, memory_values, importance_weights,
           Wq, bq, Wa, ba, Wc, bc, top_k):
    qn = _qproj(query_input, Wq, bq.reshape(1, D))
    iw3 = jnp.pad(importance_weights, (0, NB * MBLK - M)).reshape(NB, 1, MBLK)
    top_idx = _simstopk(qn, memory_keys, iw3)
    retrieved = _sc_gather(memory_values, top_idx.reshape(-1))
    r = retrieved.reshape(B, K, D)
    msk = jnp.where(jnp.arange(K) < top_k, 0.0, -1e30).astype(
        jnp.float32).reshape(1, K)
    return _attn(r, Wa.reshape(1, D), ba.reshape(1, 1), msk,
                 Wc, bc.reshape(1, D))


# speedup vs baseline: 4.2602x; 3.2538x over previous
import functools

import jax
import jax.numpy as jnp
from jax import lax
from jax.experimental import pallas as pl
from jax.experimental.pallas import tpu as pltpu
from jax.experimental.pallas import tpu_sc as plsc

B = 1024
CTRL = 1024
D = 128
M = 100000
K = 8
MBLK = 2048
NB = (M + MBLK - 1) // MBLK
NEG = -3.0e38


def _qproj_body(x_ref, wq_ref, bq_ref, o_ref):
    q = jnp.dot(x_ref[...], wq_ref[...], preferred_element_type=jnp.float32)
    q = q + bq_ref[...]
    n = jnp.sqrt(jnp.sum(q * q, axis=1, keepdims=True))
    o_ref[...] = q / jnp.maximum(n, 1e-12)


def _qproj(query_input, Wq, bq2):
    return pl.pallas_call(
        _qproj_body,
        grid=(4,),
        in_specs=[
            pl.BlockSpec((B // 4, CTRL), lambda i: (i, 0)),
            pl.BlockSpec((CTRL, D), lambda i: (0, 0)),
            pl.BlockSpec((1, D), lambda i: (0, 0)),
        ],
        out_specs=pl.BlockSpec((B // 4, D), lambda i: (i, 0)),
        out_shape=jax.ShapeDtypeStruct((B, D), jnp.float32),
    )(query_input, Wq, bq2)


BIGI = 2**30


def _extract_topk(vals, idxs, nrounds):
    tv, ti = [], []
    cur = vals
    for _ in range(nrounds):
        m = jnp.max(cur, axis=1, keepdims=True)
        hit = cur >= m
        ii = jnp.min(jnp.where(hit, idxs, BIGI), axis=1, keepdims=True)
        cur = jnp.where(idxs == ii, NEG, cur)
        tv.append(m)
        ti.append(ii)
    return jnp.concatenate(tv, axis=1), jnp.concatenate(ti, axis=1)


def _simstopk_body(qn_ref, k_ref, iw_ref, idx_ref, accv_ref, acci_ref):
    j = pl.program_id(0)
    nb = pl.num_programs(0)

    @pl.when(j == 0)
    def _():
        accv_ref[...] = jnp.full((B, K), NEG, jnp.float32)
        acci_ref[...] = jnp.full((B, K), -1, jnp.int32)

    kblk = k_ref[...]
    ss = jnp.sum(kblk * kblk, axis=1, keepdims=True)
    kn = kblk / jnp.maximum(jnp.sqrt(ss), 1e-12)
    s = lax.dot_general(qn_ref[...], kn, (((1,), (1,)), ((), ())),
                        preferred_element_type=jnp.float32)
    s = s * iw_ref[0]
    gcol = j * MBLK + lax.broadcasted_iota(jnp.int32, (B, MBLK), 1)
    s = jnp.where(gcol < M, s, NEG)

    tile_v, tile_i = _extract_topk(s, gcol, K)
    cand_v = jnp.concatenate([accv_ref[...], tile_v], axis=1)
    cand_i = jnp.concatenate([acci_ref[...], tile_i], axis=1)
    new_v, new_i = _extract_topk(cand_v, cand_i, K)
    accv_ref[...] = new_v
    acci_ref[...] = new_i

    @pl.when(j == nb - 1)
    def _():
        idx_ref[...] = new_i


def _simstopk(qn, memory_keys, iw3):
    return pl.pallas_call(
        _simstopk_body,
        grid=(NB,),
        in_specs=[
            pl.BlockSpec((B, D), lambda j: (0, 0)),
            pl.BlockSpec((MBLK, D), lambda j: (j, 0)),
            pl.BlockSpec((1, 1, MBLK), lambda j: (j, 0, 0)),
        ],
        out_specs=pl.BlockSpec((B, K), lambda j: (0, 0)),
        out_shape=jax.ShapeDtypeStruct((B, K), jnp.int32),
        scratch_shapes=[
            pltpu.VMEM((B, K), jnp.float32),
            pltpu.VMEM((B, K), jnp.int32),
        ],
        compiler_params=pltpu.CompilerParams(
            dimension_semantics=("arbitrary",)),
    )(qn, memory_keys, iw3)


def _sc_gather(memory_values, idx_flat):
    info = plsc.get_sparse_core_info()
    nw = info.num_cores * info.num_subcores
    n_idx = B * K
    b_per_w = n_idx // nw
    ch = min(128, b_per_w)
    nch = b_per_w // ch
    mesh = plsc.VectorSubcoreMesh(core_axis_name="c", subcore_axis_name="s")

    @functools.partial(
        pl.kernel,
        mesh=mesh,
        out_type=jax.ShapeDtypeStruct((n_idx, D), jnp.float32),
        scratch_types=[
            pltpu.VMEM((ch,), jnp.int32),
            pltpu.VMEM((ch, D), jnp.float32),
            pltpu.SemaphoreType.DMA,
        ],
    )
    def _gather(table_hbm, idx_hbm, out_hbm, idx_v, rows_v, sem):
        wid = lax.axis_index("s") * info.num_cores + lax.axis_index("c")
        base = wid * b_per_w
        for c in range(nch):
            off = base + c * ch
            pltpu.sync_copy(idx_hbm.at[pl.ds(off, ch)], idx_v)
            pltpu.async_copy(table_hbm.at[idx_v], rows_v, sem).wait()
            pltpu.sync_copy(rows_v, out_hbm.at[pl.ds(off, ch)])

    return _gather(memory_values, idx_flat)


def _attn_body(r_ref, wa_ref, ba_ref, msk_ref, wc_ref, bc_ref, o_ref):
    r = r_ref[...]
    wa = wa_ref[...]
    logits = jnp.sum(r * wa[None, :, :], axis=2)
    logits = logits + ba_ref[...] + msk_ref[...]
    mx = jnp.max(logits, axis=1, keepdims=True)
    e = jnp.exp(logits - mx)
    attn = e / jnp.sum(e, axis=1, keepdims=True)
    comb = jnp.sum(r * attn[:, :, None], axis=1)
    o_ref[...] = jnp.dot(comb, wc_ref[...],
                         preferred_element_type=jnp.float32) + bc_ref[...]


def _attn(r, wa2, ba2, msk, Wc, bc2):
    bb = B // 4
    return pl.pallas_call(
        _attn_body,
        grid=(4,),
        in_specs=[
            pl.BlockSpec((bb, K, D), lambda i: (i, 0, 0)),
            pl.BlockSpec((1, D), lambda i: (0, 0)),
            pl.BlockSpec((1, 1), lambda i: (0, 0)),
            pl.BlockSpec((1, K), lambda i: (0, 0)),
            pl.BlockSpec((D, D), lambda i: (0, 0)),
            pl.BlockSpec((1, D), lambda i: (0, 0)),
        ],
        out_specs=pl.BlockSpec((bb, D), lambda i: (i, 0)),
        out_shape=jax.ShapeDtypeStruct((B, D), jnp.float32),
    )(r, wa2, ba2, msk, Wc, bc2)


def kernel(query_input, memory_keys, memory_values, importance_weights,
           Wq, bq, Wa, ba, Wc, bc, top_k):
    qn = _qproj(query_input, Wq, bq.reshape(1, D))
    iw3 = jnp.pad(importance_weights, (0, NB * MBLK - M)).reshape(NB, 1, MBLK)
    top_idx = _simstopk(qn, memory_keys, iw3)
    retrieved = _sc_gather(memory_values, top_idx.reshape(-1))
    r = retrieved.reshape(B, K, D)
    msk = jnp.where(jnp.arange(K) < top_k, 0.0, -1e30).astype(
        jnp.float32).reshape(1, K)
    return _attn(r, Wa.reshape(1, D), ba.reshape(1, 1), msk,
                 Wc, bc.reshape(1, D))
